# Initial kernel scaffold; baseline (speedup 1.0000x reference)
import jax
import jax.numpy as jnp
from jax.experimental import pallas as pl


def kernel(X, table):
    raise NotImplementedError("write your pallas kernel here")


# speedup vs baseline: 1.1618x; 1.1618x over previous
import functools

import jax
import jax.numpy as jnp
from jax import lax
from jax.experimental import pallas as pl
from jax.experimental.pallas import tpu as pltpu
from jax.experimental.pallas import tpu_sc as plsc

NUM_FEATURES = 100000
EMBED_DIM = 128
BATCH = 4096
N_FIELDS = 26

_INFO = plsc.get_sparse_core_info()
_NC = _INFO.num_cores
_NS = _INFO.num_subcores
_NW = _NC * _NS

_B_TOTAL = BATCH * N_FIELDS
_B_PER_W = _B_TOTAL // _NW
_CHUNK = 128
_N_CHUNKS = _B_PER_W // _CHUNK


def _sc_gather(idx, table):
    mesh = plsc.VectorSubcoreMesh(core_axis_name="c", subcore_axis_name="s")

    @functools.partial(
        pl.kernel,
        out_type=jax.ShapeDtypeStruct((_B_TOTAL, EMBED_DIM), jnp.float32),
        mesh=mesh,
        scratch_types=[
            pltpu.VMEM((_N_CHUNKS, _CHUNK), jnp.int32),
            pltpu.VMEM((_CHUNK, EMBED_DIM), jnp.float32),
            pltpu.SemaphoreType.DMA,
        ],
    )
    def k(idx_hbm, table_hbm, out_hbm, idx_v, buf, gsem):
        wid = lax.axis_index("s") * _NC + lax.axis_index("c")
        base = wid * _B_PER_W
        pltpu.sync_copy(idx_hbm.at[wid], idx_v)

        def step(j, carry):
            pltpu.async_copy(table_hbm.at[idx_v.at[j]], buf, gsem).wait()
            pltpu.sync_copy(buf, out_hbm.at[pl.ds(base + j * _CHUNK, _CHUNK)])
            return carry

        lax.fori_loop(0, _N_CHUNKS, step, 0)

    return k(idx, table)


def kernel(X, table):
    idx = X.reshape(_NW, _N_CHUNKS, _CHUNK).astype(jnp.int32)
    out = _sc_gather(idx, table)
    return out.reshape(BATCH, N_FIELDS, EMBED_DIM)


# speedup vs baseline: 1.2123x; 1.0435x over previous
import functools

import jax
import jax.numpy as jnp
from jax import lax
from jax.experimental import pallas as pl
from jax.experimental.pallas import tpu as pltpu
from jax.experimental.pallas import tpu_sc as plsc

NUM_FEATURES = 100000
EMBED_DIM = 128
BATCH = 4096
N_FIELDS = 26

_INFO = plsc.get_sparse_core_info()
_NC = _INFO.num_cores
_NS = _INFO.num_subcores
_NW = _NC * _NS

_B_TOTAL = BATCH * N_FIELDS
_B_PER_W = _B_TOTAL // _NW
_CHUNK = 128
_N_CHUNKS = _B_PER_W // _CHUNK


def _sc_gather(idx, table):
    mesh = plsc.VectorSubcoreMesh(core_axis_name="c", subcore_axis_name="s")

    @functools.partial(
        pl.kernel,
        out_type=jax.ShapeDtypeStruct((_B_TOTAL, EMBED_DIM), jnp.float32),
        mesh=mesh,
        scratch_types=[
            pltpu.VMEM((_N_CHUNKS, _CHUNK), jnp.int32),
            pltpu.VMEM((_CHUNK, EMBED_DIM), jnp.float32),
            pltpu.VMEM((_CHUNK, EMBED_DIM), jnp.float32),
            pltpu.SemaphoreType.DMA,
            pltpu.SemaphoreType.DMA,
            pltpu.SemaphoreType.DMA,
            pltpu.SemaphoreType.DMA,
        ],
    )
    def k(idx_hbm, table_hbm, out_hbm, idx_v, buf0, buf1, gs0, gs1, os0, os1):
        wid = lax.axis_index("s") * _NC + lax.axis_index("c")
        base = wid * _B_PER_W
        pltpu.sync_copy(idx_hbm.at[wid], idx_v)

        bufs = (buf0, buf1)
        gsems = (gs0, gs1)
        osems = (os0, os1)

        def gather_start(c, b):
            pltpu.async_copy(table_hbm.at[idx_v.at[c]], bufs[b], gsems[b])

        def gather_wait(b):
            pltpu.make_async_copy(
                table_hbm.at[idx_v.at[0]], bufs[b], gsems[b]
            ).wait()

        def out_start(c, b):
            pltpu.async_copy(
                bufs[b], out_hbm.at[pl.ds(base + c * _CHUNK, _CHUNK)], osems[b]
            )

        def out_wait(b):
            pltpu.make_async_copy(
                bufs[b], out_hbm.at[pl.ds(base, _CHUNK)], osems[b]
            ).wait()

        gather_start(0, 0)
        gather_wait(0)
        out_start(0, 0)
        gather_start(1, 1)

        def pair(p, carry):
            c = 2 * p + 1
            gather_wait(1)
            out_start(c, 1)
            out_wait(0)
            gather_start(c + 1, 0)
            gather_wait(0)
            out_start(c + 1, 0)
            out_wait(1)
            gather_start(c + 2, 1)
            return carry

        lax.fori_loop(0, (_N_CHUNKS - 2) // 2, pair, 0)

        gather_wait(1)
        out_start(_N_CHUNKS - 1, 1)
        out_wait(0)
        out_wait(1)

    return k(idx, table)


def kernel(X, table):
    idx = X.reshape(_NW, _N_CHUNKS, _CHUNK).astype(jnp.int32)
    out = _sc_gather(idx, table)
    return out.reshape(BATCH, N_FIELDS, EMBED_DIM)


# speedup vs baseline: 1.2847x; 1.0598x over previous
import functools

import jax
import jax.numpy as jnp
from jax import lax
from jax.experimental import pallas as pl
from jax.experimental.pallas import tpu as pltpu
from jax.experimental.pallas import tpu_sc as plsc

NUM_FEATURES = 100000
EMBED_DIM = 128
BATCH = 4096
N_FIELDS = 26

_INFO = plsc.get_sparse_core_info()
_NC = _INFO.num_cores
_NS = _INFO.num_subcores
_NW = _NC * _NS

_B_TOTAL = BATCH * N_FIELDS
_B_PER_W = _B_TOTAL // _NW
_CHUNK = 128
_N_CHUNKS = _B_PER_W // _CHUNK


def _sc_gather(idx, table):
    mesh = plsc.VectorSubcoreMesh(core_axis_name="c", subcore_axis_name="s")

    @functools.partial(
        pl.kernel,
        out_type=jax.ShapeDtypeStruct((_B_TOTAL, EMBED_DIM), jnp.float32),
        mesh=mesh,
        scratch_types=[
            pltpu.VMEM((_N_CHUNKS, _CHUNK), jnp.int32),
            pltpu.VMEM((_CHUNK, EMBED_DIM), jnp.float32),
            pltpu.VMEM((_CHUNK, EMBED_DIM), jnp.float32),
            pltpu.VMEM((_CHUNK, EMBED_DIM), jnp.float32),
            pltpu.VMEM((_CHUNK, EMBED_DIM), jnp.float32),
            pltpu.SemaphoreType.DMA,
            pltpu.SemaphoreType.DMA,
            pltpu.SemaphoreType.DMA,
            pltpu.SemaphoreType.DMA,
            pltpu.SemaphoreType.DMA,
            pltpu.SemaphoreType.DMA,
            pltpu.SemaphoreType.DMA,
            pltpu.SemaphoreType.DMA,
        ],
    )
    def k(idx_hbm, table_hbm, out_hbm, idx_v, buf0, buf1, buf2, buf3,
          gs0, gs1, gs2, gs3, os0, os1, os2, os3):
        wid = lax.axis_index("s") * _NC + lax.axis_index("c")
        base = wid * _B_PER_W
        pltpu.sync_copy(idx_hbm.at[wid], idx_v)

        bufs = (buf0, buf1, buf2, buf3)
        gsems = (gs0, gs1, gs2, gs3)
        osems = (os0, os1, os2, os3)

        def gather_start(c, b):
            pltpu.async_copy(table_hbm.at[idx_v.at[c]], bufs[b], gsems[b])

        def gather_wait(b):
            pltpu.make_async_copy(
                table_hbm.at[idx_v.at[0]], bufs[b], gsems[b]
            ).wait()

        def out_start(c, b):
            pltpu.async_copy(
                bufs[b], out_hbm.at[pl.ds(base + c * _CHUNK, _CHUNK)], osems[b]
            )

        def out_wait(b):
            pltpu.make_async_copy(
                bufs[b], out_hbm.at[pl.ds(base, _CHUNK)], osems[b]
            ).wait()

        gather_start(0, 0)
        gather_start(1, 1)
        gather_wait(0)
        out_start(0, 0)
        gather_start(2, 2)
        gather_wait(1)
        out_start(1, 1)
        gather_start(3, 3)

        def quad(p, carry):
            for i in range(4):
                c = 4 * p + 2 + i
                b = (2 + i) % 4
                bn = (b + 2) % 4
                gather_wait(b)
                out_start(c, b)
                out_wait(bn)
                gather_start(c + 2, bn)
            return carry

        lax.fori_loop(0, (_N_CHUNKS - 6) // 4, quad, 0)

        gather_wait(2)
        out_start(_N_CHUNKS - 4, 2)
        out_wait(0)
        gather_start(_N_CHUNKS - 2, 0)
        gather_wait(3)
        out_start(_N_CHUNKS - 3, 3)
        out_wait(1)
        gather_start(_N_CHUNKS - 1, 1)
        gather_wait(0)
        out_start(_N_CHUNKS - 2, 0)
        gather_wait(1)
        out_start(_N_CHUNKS - 1, 1)
        out_wait(2)
        out_wait(3)
        out_wait(0)
        out_wait(1)

    return k(idx, table)


def kernel(X, table):
    idx = X.reshape(_NW, _N_CHUNKS, _CHUNK).astype(jnp.int32)
    out = _sc_gather(idx, table)
    return out.reshape(BATCH, N_FIELDS, EMBED_DIM)


# speedup vs baseline: 1.2904x; 1.0044x over previous
import functools

import jax
import jax.numpy as jnp
from jax import lax
from jax.experimental import pallas as pl
from jax.experimental.pallas import tpu as pltpu
from jax.experimental.pallas import tpu_sc as plsc

NUM_FEATURES = 100000
EMBED_DIM = 128
BATCH = 4096
N_FIELDS = 26

_INFO = plsc.get_sparse_core_info()
_NC = _INFO.num_cores
_NS = _INFO.num_subcores
_NW = _NC * _NS

_B_TOTAL = BATCH * N_FIELDS
_B_PER_W = _B_TOTAL // _NW

_CHUNK = 256
_N_CHUNKS = _B_PER_W // _CHUNK
_NB = 3
_LA = 2


def _sc_gather(idx, table):
    mesh = plsc.VectorSubcoreMesh(core_axis_name="c", subcore_axis_name="s")

    @functools.partial(
        pl.kernel,
        out_type=jax.ShapeDtypeStruct((_B_TOTAL, EMBED_DIM), jnp.float32),
        mesh=mesh,
        scratch_types=(
            [pltpu.VMEM((_B_PER_W,), jnp.int32)]
            + [pltpu.VMEM((_CHUNK, EMBED_DIM), jnp.float32)] * _NB
            + [pltpu.SemaphoreType.DMA] * (2 * _NB)
        ),
    )
    def k(idx_hbm, table_hbm, out_hbm, idx_v, *rest):
        bufs = rest[:_NB]
        gsems = rest[_NB:2 * _NB]
        osems = rest[2 * _NB:]

        wid = lax.axis_index("s") * _NC + lax.axis_index("c")
        base = wid * _B_PER_W
        pltpu.sync_copy(idx_hbm.at[wid], idx_v)

        def gather_start(c, b):
            pltpu.async_copy(
                table_hbm.at[idx_v.at[pl.ds(c * _CHUNK, _CHUNK)]],
                bufs[b], gsems[b],
            )

        def gather_wait(b):
            pltpu.make_async_copy(
                table_hbm.at[idx_v.at[pl.ds(0, _CHUNK)]], bufs[b], gsems[b]
            ).wait()

        def out_start(c, b):
            pltpu.async_copy(
                bufs[b], out_hbm.at[pl.ds(base + c * _CHUNK, _CHUNK)], osems[b]
            )

        def out_wait(b):
            pltpu.make_async_copy(
                bufs[b], out_hbm.at[pl.ds(base, _CHUNK)], osems[b]
            ).wait()

        out_pending = [False] * _NB

        def drain_out(b):
            if out_pending[b]:
                out_wait(b)
                out_pending[b] = False

        for c in range(min(_LA, _N_CHUNKS)):
            gather_start(c, c % _NB)
        for c in range(_N_CHUNKS):
            b = c % _NB
            gather_wait(b)
            out_start(c, b)
            out_pending[b] = True
            nc = c + _LA
            if nc < _N_CHUNKS:
                bn = nc % _NB
                drain_out(bn)
                gather_start(nc, bn)
        for b in range(_NB):
            drain_out(b)

    return k(idx, table)


def kernel(X, table):
    idx = X.reshape(_NW, _B_PER_W).astype(jnp.int32)
    out = _sc_gather(idx, table)
    return out.reshape(BATCH, N_FIELDS, EMBED_DIM)


# speedup vs baseline: 1.9625x; 1.5208x over previous
import functools

import jax
import jax.numpy as jnp
from jax import lax
from jax.experimental import pallas as pl
from jax.experimental.pallas import tpu as pltpu
from jax.experimental.pallas import tpu_sc as plsc

NUM_FEATURES = 100000
EMBED_DIM = 128
BATCH = 4096
N_FIELDS = 26

_INFO = plsc.get_sparse_core_info()
_NC = _INFO.num_cores
_NS = _INFO.num_subcores
_NW = _NC * _NS

_BATCH_PER_W = BATCH // _NW
_NB = 8
_LA = 4


def _sc_gather(idx, table):
    mesh = plsc.VectorSubcoreMesh(core_axis_name="c", subcore_axis_name="s")

    @functools.partial(
        pl.kernel,
        out_type=jax.ShapeDtypeStruct((BATCH, N_FIELDS, EMBED_DIM), jnp.float32),
        mesh=mesh,
        scratch_types=(
            [pltpu.VMEM((_BATCH_PER_W, N_FIELDS), jnp.int32)]
            + [pltpu.VMEM((N_FIELDS, EMBED_DIM), jnp.float32)] * _NB
            + [pltpu.SemaphoreType.DMA] * (2 * _NB)
        ),
    )
    def k(idx_hbm, table_hbm, out_hbm, idx_v, *rest):
        bufs = rest[:_NB]
        gsems = rest[_NB:2 * _NB]
        osems = rest[2 * _NB:]

        wid = lax.axis_index("s") * _NC + lax.axis_index("c")
        base = wid * _BATCH_PER_W
        pltpu.sync_copy(idx_hbm.at[wid], idx_v)

        def gather_start(kk, b):
            pltpu.async_copy(table_hbm.at[idx_v.at[kk]], bufs[b], gsems[b])

        def gather_wait(b):
            pltpu.make_async_copy(
                table_hbm.at[idx_v.at[0]], bufs[b], gsems[b]
            ).wait()

        def out_start(kk, b):
            pltpu.async_copy(bufs[b], out_hbm.at[base + kk], osems[b])

        def out_wait(b):
            pltpu.make_async_copy(bufs[b], out_hbm.at[base], osems[b]).wait()

        for kk in range(_LA):
            gather_start(kk, kk)

        def body(p, carry):
            for b in range(_NB):
                kk = p * _NB + b
                gather_wait(b)
                out_start(kk, b)
                bn = (b + _LA) % _NB
                if b < _LA:
                    @pl.when(p >= 1)
                    def _():
                        out_wait(bn)
                    gather_start(kk + _LA, bn)
                else:
                    @pl.when(p < _BATCH_PER_W // _NB - 1)
                    def _():
                        out_wait(bn)
                        gather_start(kk + _LA, bn)
            return carry

        lax.fori_loop(0, _BATCH_PER_W // _NB, body, 0)

        for b in range(_NB):
            out_wait(b)

    return k(idx, table)


def kernel(X, table):
    idx = X.reshape(_NW, _BATCH_PER_W, N_FIELDS).astype(jnp.int32)
    out = _sc_gather(idx, table)
    return out


# speedup vs baseline: 2.0511x; 1.0452x over previous
import functools

import jax
import jax.numpy as jnp
from jax import lax
from jax.experimental import pallas as pl
from jax.experimental.pallas import tpu as pltpu
from jax.experimental.pallas import tpu_sc as plsc

NUM_FEATURES = 100000
EMBED_DIM = 128
BATCH = 4096
N_FIELDS = 26

_INFO = plsc.get_sparse_core_info()
_NC = _INFO.num_cores
_NS = _INFO.num_subcores
_NW = _NC * _NS

_BATCH_PER_W = BATCH // _NW
_BPC = 4
_ROWS_PER_C = _BPC * N_FIELDS
_N_CHUNKS = _BATCH_PER_W // _BPC
_NB = 8
_LA = 4


def _sc_gather(idx, table):
    mesh = plsc.VectorSubcoreMesh(core_axis_name="c", subcore_axis_name="s")

    @functools.partial(
        pl.kernel,
        out_type=jax.ShapeDtypeStruct((BATCH, N_FIELDS, EMBED_DIM), jnp.float32),
        mesh=mesh,
        scratch_types=(
            [pltpu.VMEM((_BATCH_PER_W * N_FIELDS,), jnp.int32)]
            + [pltpu.VMEM((_ROWS_PER_C, EMBED_DIM), jnp.float32)] * _NB
            + [pltpu.SemaphoreType.DMA] * (2 * _NB)
        ),
    )
    def k(idx_hbm, table_hbm, out_hbm, idx_v, *rest):
        bufs = rest[:_NB]
        gsems = rest[_NB:2 * _NB]
        osems = rest[2 * _NB:]

        wid = lax.axis_index("s") * _NC + lax.axis_index("c")
        base = wid * _BATCH_PER_W
        pltpu.sync_copy(idx_hbm.at[wid], idx_v)

        def gather_start(c, b):
            pltpu.async_copy(
                table_hbm.at[idx_v.at[pl.ds(c * _ROWS_PER_C, _ROWS_PER_C)]],
                bufs[b], gsems[b],
            )

        def gather_wait(b):
            pltpu.make_async_copy(
                table_hbm.at[idx_v.at[pl.ds(0, _ROWS_PER_C)]], bufs[b], gsems[b]
            ).wait()

        def out_start(c, b):
            for j in range(_BPC):
                pltpu.async_copy(
                    bufs[b].at[pl.ds(j * N_FIELDS, N_FIELDS)],
                    out_hbm.at[base + c * _BPC + j],
                    osems[b],
                )

        def out_wait(b):
            for _ in range(_BPC):
                pltpu.make_async_copy(
                    bufs[b].at[pl.ds(0, N_FIELDS)], out_hbm.at[base], osems[b]
                ).wait()

        for c in range(_LA):
            gather_start(c, c)

        def body(p, carry):
            for b in range(_NB):
                c = p * _NB + b
                gather_wait(b)
                out_start(c, b)
                bn = (b + _LA) % _NB
                if b < _LA:
                    @pl.when(p >= 1)
                    def _():
                        out_wait(bn)
                    gather_start(c + _LA, bn)
                else:
                    @pl.when(p < _N_CHUNKS // _NB - 1)
                    def _():
                        out_wait(bn)
                        gather_start(c + _LA, bn)
            return carry

        lax.fori_loop(0, _N_CHUNKS // _NB, body, 0)

        for b in range(_NB):
            out_wait(b)

    return k(idx, table)


def kernel(X, table):
    idx = X.reshape(_NW, _BATCH_PER_W * N_FIELDS).astype(jnp.int32)
    out = _sc_gather(idx, table)
    return out
